# Initial kernel scaffold; baseline (speedup 1.0000x reference)
import jax
import jax.numpy as jnp
from jax.experimental import pallas as pl


def kernel(inputs, weight):
    raise NotImplementedError("write your pallas kernel here")


# speedup vs baseline: 1.8625x; 1.8625x over previous
import functools

import jax
import jax.numpy as jnp
from jax import lax
from jax.experimental import pallas as pl
from jax.experimental.pallas import tpu as pltpu
from jax.experimental.pallas import tpu_sc as plsc

VOCAB_SIZE = 1000000
HIDDEN = 128

NC = 2
NS = 16
NW = NC * NS

CHUNK = 128
B_TOTAL = 4096 * 200
B_PER_W = B_TOTAL // NW
NCHUNK = B_PER_W // CHUNK


def _mesh():
    return plsc.VectorSubcoreMesh(
        core_axis_name="c", subcore_axis_name="s", num_cores=NC, num_subcores=NS
    )


@functools.partial(
    pl.kernel,
    out_type=jax.ShapeDtypeStruct((NW, NCHUNK, CHUNK, HIDDEN), jnp.float32),
    mesh=_mesh(),
    scratch_types=[
        pltpu.VMEM((NCHUNK, CHUNK), jnp.int32),
        pltpu.VMEM((CHUNK, HIDDEN), jnp.float32),
        pltpu.VMEM((CHUNK, HIDDEN), jnp.float32),
        pltpu.SemaphoreType.DMA,
        pltpu.SemaphoreType.DMA,
    ],
)
def _gather_kernel(idx_hbm, table_hbm, out_hbm, idx_v, rows0, rows1, sem0, sem1):
    wid = lax.axis_index("s") * NC + lax.axis_index("c")

    pltpu.sync_copy(idx_hbm.at[wid], idx_v)

    def start(j, buf, sem):
        pltpu.make_async_copy(table_hbm.at[idx_v.at[j]], buf, sem).start()

    def finish(j, buf, sem):
        pltpu.make_async_copy(table_hbm.at[idx_v.at[j]], buf, sem).wait()
        pltpu.sync_copy(buf, out_hbm.at[wid, j])

    start(0, rows0, sem0)

    def pair(p, _):
        j0 = 2 * p
        start(j0 + 1, rows1, sem1)
        finish(j0, rows0, sem0)

        @pl.when(p + 1 < NCHUNK // 2)
        def _():
            start(j0 + 2, rows0, sem0)

        finish(j0 + 1, rows1, sem1)
        return 0

    lax.fori_loop(0, NCHUNK // 2, pair, 0)


def kernel(inputs, weight):
    idx = inputs.astype(jnp.int32).reshape(NW, NCHUNK, CHUNK)
    out = _gather_kernel(idx, weight)
    return out.reshape(4096, 200, HIDDEN)
